# Initial kernel scaffold; baseline (speedup 1.0000x reference)
import functools

import jax
import jax.numpy as jnp
from jax import lax
from jax.experimental import pallas as pl
from jax.experimental.pallas import tpu as pltpu
from jax.experimental.pallas import tpu_sc as plsc

_VOCAB = 100000
_D = 64
_B = 4096
_S = 200
_N = _B * _S
_LN = 128
_ROWS = _N // _LN
_NC = 2
_NS = 16
_NW = _NC * _NS
_ROWS_PER_W = _ROWS // _NW
_K = 4
_NCHUNK = _ROWS_PER_W // _K

_mesh = plsc.VectorSubcoreMesh(core_axis_name="c", subcore_axis_name="s")


@functools.partial(
    pl.kernel,
    out_type=jax.ShapeDtypeStruct((_ROWS, _LN, _D), jnp.float32),
    mesh=_mesh,
    scratch_types=[
        pltpu.VMEM((_K, _LN), jnp.int32),
        pltpu.VMEM((_K, _LN, _D), jnp.float32),
        pltpu.SemaphoreType.DMA,
    ],
)
def _sc_gather(table_hbm, ids_hbm, out_hbm, idx_v, rows_v, sem):
    wid = lax.axis_index("s") * _NC + lax.axis_index("c")
    base = wid * _ROWS_PER_W

    def step(g, carry):
        r0 = base + g * _K
        pltpu.sync_copy(ids_hbm.at[pl.ds(r0, _K)], idx_v)
        cps = [
            pltpu.async_copy(table_hbm.at[idx_v.at[j]], rows_v.at[j], sem)
            for j in range(_K)
        ]
        for cp in cps:
            cp.wait()
        pltpu.sync_copy(rows_v, out_hbm.at[pl.ds(r0, _K)])
        return carry

    lax.fori_loop(0, _NCHUNK, step, 0)


def kernel(ipa_ids, lengths, table):
    ids = ipa_ids.astype(jnp.int32).reshape(_ROWS, _LN)
    out = _sc_gather(table, ids)
    return (out.reshape(_B, _S, _D), lengths)


# speedup vs baseline: 3.9922x; 3.9922x over previous
import functools

import jax
import jax.numpy as jnp
from jax import lax
from jax.experimental import pallas as pl
from jax.experimental.pallas import tpu as pltpu
from jax.experimental.pallas import tpu_sc as plsc

_VOCAB = 100000
_D = 64
_B = 4096
_S = 200
_N = _B * _S
_LN = 128
_ROWS = _N // _LN
_NC = 2
_NS = 16
_NW = _NC * _NS
_ROWS_PER_W = _ROWS // _NW
_K = 4
_NCHUNK = _ROWS_PER_W // _K

_mesh = plsc.VectorSubcoreMesh(core_axis_name="c", subcore_axis_name="s")


@functools.partial(
    pl.kernel,
    out_type=jax.ShapeDtypeStruct((_ROWS, _LN, _D), jnp.float32),
    mesh=_mesh,
    scratch_types=[
        pltpu.VMEM((_K, _LN), jnp.int32),
        pltpu.VMEM((_K, _LN, _D), jnp.float32),
        pltpu.SemaphoreType.DMA,
    ],
    compiler_params=pltpu.CompilerParams(use_tc_tiling_on_sc=False),
)
def _sc_gather(table_hbm, ids_hbm, out_hbm, idx_v, rows_v, sem):
    wid = lax.axis_index("s") * _NC + lax.axis_index("c")
    base = wid * _ROWS_PER_W

    def step(g, carry):
        r0 = base + g * _K
        pltpu.sync_copy(ids_hbm.at[pl.ds(r0, _K)], idx_v)
        cps = [
            pltpu.async_copy(table_hbm.at[idx_v.at[j]], rows_v.at[j], sem)
            for j in range(_K)
        ]
        for cp in cps:
            cp.wait()
        pltpu.sync_copy(rows_v, out_hbm.at[pl.ds(r0, _K)])
        return carry

    lax.fori_loop(0, _NCHUNK, step, 0)


def kernel(ipa_ids, lengths, table):
    ids = ipa_ids.astype(jnp.int32).reshape(_ROWS, _LN)
    out = _sc_gather(table, ids)
    return (out.reshape(_B, _S, _D), lengths)


# speedup vs baseline: 4.2918x; 1.0750x over previous
import functools

import jax
import jax.numpy as jnp
from jax import lax
from jax.experimental import pallas as pl
from jax.experimental.pallas import tpu as pltpu
from jax.experimental.pallas import tpu_sc as plsc

_VOCAB = 100000
_D = 64
_B = 4096
_S = 200
_N = _B * _S
_LN = 128
_ROWS = _N // _LN
_NC = 2
_NS = 16
_NW = _NC * _NS
_RPW = _ROWS // _NW
_K = 2
_NBUF = 4
_NCHUNK = _RPW // _K

_mesh = plsc.VectorSubcoreMesh(core_axis_name="c", subcore_axis_name="s")


@functools.partial(
    pl.kernel,
    out_type=jax.ShapeDtypeStruct((_ROWS, _LN, _D), jnp.float32),
    mesh=_mesh,
    scratch_types=[
        pltpu.VMEM((_RPW, _LN), jnp.int32),
        pltpu.VMEM((_NBUF, _K, _LN, _D), jnp.float32),
    ]
    + [pltpu.SemaphoreType.DMA] * (2 * _NBUF),
    compiler_params=pltpu.CompilerParams(use_tc_tiling_on_sc=False),
)
def _sc_gather(table_hbm, ids_hbm, out_hbm, idx_all, rows_v, *sems):
    gsems = sems[:_NBUF]
    wsems = sems[_NBUF:]
    wid = lax.axis_index("s") * _NC + lax.axis_index("c")
    base = wid * _RPW

    def fire_gathers(g, b):
        for j in range(_K):
            pltpu.make_async_copy(
                table_hbm.at[idx_all.at[g * _K + j]],
                rows_v.at[b, j],
                gsems[b],
            ).start()

    def wait_gathers(b):
        pltpu.make_async_copy(
            out_hbm.at[pl.ds(0, _K)], rows_v.at[b], gsems[b]
        ).wait()

    def fire_write(g, b):
        pltpu.make_async_copy(
            rows_v.at[b], out_hbm.at[pl.ds(base + g * _K, _K)], wsems[b]
        ).start()

    def wait_write(b):
        pltpu.make_async_copy(
            rows_v.at[b], out_hbm.at[pl.ds(0, _K)], wsems[b]
        ).wait()

    pltpu.sync_copy(ids_hbm.at[pl.ds(base, _RPW)], idx_all)

    fire_gathers(0, 0)
    fire_gathers(1, 1)
    for g in (0, 1):
        wait_gathers(g)
        fire_write(g, g)
        fire_gathers(g + 2, (g + 2) % _NBUF)

    def body(t, carry):
        g0 = 2 + _NBUF * t
        for u in range(_NBUF):
            g = g0 + u
            b = (2 + u) % _NBUF
            bn = u % _NBUF
            wait_gathers(b)
            fire_write(g, b)
            wait_write(bn)
            fire_gathers(g + 2, bn)
        return carry

    lax.fori_loop(0, (_NCHUNK - 4) // _NBUF, body, 0)

    for g in (_NCHUNK - 2, _NCHUNK - 1):
        b = g % _NBUF
        wait_gathers(b)
        fire_write(g, b)

    for b in range(_NBUF):
        wait_write(b)


def kernel(ipa_ids, lengths, table):
    ids = ipa_ids.astype(jnp.int32).reshape(_ROWS, _LN)
    out = _sc_gather(table, ids)
    return (out.reshape(_B, _S, _D), lengths)
